# Initial kernel scaffold; baseline (speedup 1.0000x reference)
import jax
import jax.numpy as jnp
from jax.experimental import pallas as pl


def kernel(X, all_rep, n_recos):
    raise NotImplementedError("write your pallas kernel here")


# speedup vs baseline: 1.1533x; 1.1533x over previous
import functools

import jax
import jax.numpy as jnp
from jax.experimental import pallas as pl
from jax.experimental.pallas import tpu as pltpu

VOCAB = 100000
D = 64
B = 1024
L = 50
K = 500

ROW_TILE = 256
COL_TILE = 2048
PAD_COLS = ((VOCAB + COL_TILE - 1) // COL_TILE) * COL_TILE


def _score_body(x_ref, xagg_ref, item_ref, out_ref):
    j = pl.program_id(0)
    xagg = xagg_ref[...]
    item = item_ref[...]
    scores = jax.lax.dot_general(
        xagg, item, (((1,), (1,)), ((), ())),
        preferred_element_type=jnp.float32,
    )
    col = j * COL_TILE + jax.lax.broadcasted_iota(jnp.int32, (1, COL_TILE), 1)
    xids = x_ref[...] - 1
    for l in range(L):
        xid = xids[:, l][:, None]
        scores = jnp.where(col == xid, -1.0e3, scores)
    scores = jnp.where(col >= VOCAB, -jnp.inf, scores)
    out_ref[...] = scores


@functools.partial(jax.jit, static_argnames=())
def kernel(X, all_rep, n_recos):
    X_rep = jnp.take(all_rep, X, axis=0)
    X_agg = jnp.mean(X_rep, axis=1)
    item_rep = all_rep[1:-1]
    item_pad = jnp.pad(item_rep, ((0, PAD_COLS - VOCAB), (0, 0)))

    grid = (PAD_COLS // COL_TILE, B // ROW_TILE)
    scores = pl.pallas_call(
        _score_body,
        grid=grid,
        in_specs=[
            pl.BlockSpec((ROW_TILE, L), lambda j, i: (i, 0)),
            pl.BlockSpec((ROW_TILE, D), lambda j, i: (i, 0)),
            pl.BlockSpec((COL_TILE, D), lambda j, i: (j, 0)),
        ],
        out_specs=pl.BlockSpec((ROW_TILE, COL_TILE), lambda j, i: (i, j)),
        out_shape=jax.ShapeDtypeStruct((B, PAD_COLS), jnp.float32),
    )(X, X_agg, item_pad)

    scores = scores + (jnp.asarray(n_recos) - K).astype(scores.dtype)
    vals, idx = jax.lax.top_k(scores, K)
    return vals, idx


# speedup vs baseline: 2.0699x; 1.7948x over previous
import functools

import jax
import jax.numpy as jnp
from jax.experimental import pallas as pl
from jax.experimental.pallas import tpu as pltpu

MININT = -(2**31)


def _key_fwd(s):
    b = jax.lax.bitcast_convert_type(s, jnp.int32)
    return jnp.where(b >= 0, b, jnp.bitwise_not(b) ^ MININT)


def _key_inv(k):
    b = jnp.where(k >= 0, k, jnp.bitwise_not(k ^ MININT))
    return jax.lax.bitcast_convert_type(b, jnp.float32)


def _make_kernel(vocab, d, b, l, k, row_tile, col_tile, cap):
    padc = ((vocab + col_tile - 1) // col_tile) * col_tile
    nt = padc // col_tile
    chunk = 128
    nch = padc // chunk
    slots = 16
    cpt = (col_tile // chunk) * slots
    nsl = nch * slots
    sl_bits = max(1, (nsl - 1).bit_length())
    rk_chunk = 64

    def body(x_ref, xagg_ref, item_ref, vals_ref, idx_ref,
             ks_ref, d_ref, ks2_ref, co2_ref):
        j = pl.program_id(1)
        scores = jax.lax.dot_general(
            xagg_ref[...], item_ref[...], (((1,), (1,)), ((), ())),
            preferred_element_type=jnp.float32,
        )
        col = j * col_tile + jax.lax.broadcasted_iota(jnp.int32, (1, col_tile), 1)
        xids = x_ref[...] - 1
        for ll in range(l):
            scores = jnp.where(col == xids[:, ll][:, None], -1.0e3, scores)
        scores = jnp.where(col >= vocab, -jnp.inf, scores)
        ks_ref[:, pl.ds(j * col_tile, col_tile)] = _key_fwd(scores)

        @pl.when(j == nt - 1)
        def _():
            lane_t = jax.lax.broadcasted_iota(jnp.int32, (1, col_tile), 1)
            lane_c = lane_t % chunk
            kk = jnp.int32(k)

            def bit_body(bi, t):
                trial = t + (jnp.int32(1) << (31 - bi))

                def tile_body(ti, cnt):
                    kst = ks_ref[:, pl.ds(ti * col_tile, col_tile)]
                    return cnt + jnp.sum((kst >= trial).astype(jnp.int32),
                                         axis=1, keepdims=True)

                cnt = jax.lax.fori_loop(
                    0, nt, tile_body,
                    jnp.zeros((row_tile, 1), jnp.int32))
                return jnp.where(cnt >= kk, trial, t)

            t = jax.lax.fori_loop(
                0, 32, bit_body,
                jnp.full((row_tile, 1), MININT, jnp.int32))

            def init_body(ti, _):
                kst = ks_ref[:, pl.ds(ti * col_tile, col_tile)]
                sel = kst >= t
                p = sel.astype(jnp.int32)
                for bbit in range(7):
                    s = 1 << bbit
                    p = p + jnp.where(lane_c >= s, jnp.roll(p, s, axis=1), 0)
                dist = lane_c - (p - 1)
                d_ref[:, pl.ds(ti * col_tile, col_tile)] = jnp.where(
                    sel, (lane_c << 8) | dist, -1)
                return _

            jax.lax.fori_loop(0, nt, init_body, jnp.int32(0))

            def comp_body(ti, _):
                ksv = ks_ref[:, pl.ds(ti * col_tile, col_tile)]
                dv = d_ref[:, pl.ds(ti * col_tile, col_tile)]
                for bbit in range(7):
                    s = 1 << bbit
                    kM = jnp.roll(ksv, -s, axis=1)
                    dM = jnp.roll(dv, -s, axis=1)
                    guard = lane_c <= (chunk - 1 - s)
                    take = guard & (dM >= 0) & ((dM & s) != 0)
                    moved = (dv >= 0) & ((dv & s) != 0)
                    ksv = jnp.where(take, kM, ksv)
                    dv = jnp.where(take, dM - s,
                                   jnp.where(moved, -1, dv))
                ks3 = ksv.reshape(row_tile, col_tile // chunk, chunk)
                d3 = dv.reshape(row_tile, col_tile // chunk, chunk)
                kslot = ks3[:, :, :slots].reshape(row_tile, cpt)
                dslot = d3[:, :, :slots].reshape(row_tile, cpt)
                valid = (dslot >= 0) & ((dslot & 127) == 0)
                chk = jax.lax.broadcasted_iota(
                    jnp.int32, (1, cpt), 1) // slots
                colv = ti * col_tile + chk * chunk + (dslot >> 8)
                ks2_ref[:, pl.ds(ti * cpt, cpt)] = kslot
                co2_ref[:, pl.ds(ti * cpt, cpt)] = jnp.where(valid, colv, -1)
                return _

            jax.lax.fori_loop(0, nt, comp_body, jnp.int32(0))

            lane2 = jax.lax.broadcasted_iota(jnp.int32, (1, nsl), 1)
            v2 = (co2_ref[...] >= 0).astype(jnp.int32)
            p2 = v2
            for bbit in range(sl_bits):
                s = 1 << bbit
                p2 = p2 + jnp.where(lane2 >= s, jnp.roll(p2, s, axis=1), 0)
            dv = jnp.where(v2 != 0, lane2 - (p2 - 1), -1)
            ksv = ks2_ref[...]
            cov = co2_ref[...]
            for bbit in range(sl_bits):
                s = 1 << bbit
                kM = jnp.roll(ksv, -s, axis=1)
                cM = jnp.roll(cov, -s, axis=1)
                dM = jnp.roll(dv, -s, axis=1)
                guard = lane2 <= (nsl - 1 - s)
                take = guard & (dM >= 0) & ((dM & s) != 0)
                moved = (dv >= 0) & ((dv & s) != 0)
                ksv = jnp.where(take, kM, ksv)
                cov = jnp.where(take, cM, cov)
                dv = jnp.where(take, dM - s, jnp.where(moved, -1, dv))
            ck = ksv[:, :cap]
            cc = cov[:, :cap]
            cv = dv[:, :cap] == 0

            ckc = jnp.where(cv, ck, MININT)
            ccc = jnp.where(cv, cc, jnp.int32(2**30))
            rank = jnp.zeros((row_tile, cap), jnp.int32)
            for jc in range(cap // rk_chunk):
                kj = ckc[:, jc * rk_chunk:(jc + 1) * rk_chunk]
                cj = ccc[:, jc * rk_chunk:(jc + 1) * rk_chunk]
                gt = ((kj[:, None, :] > ckc[:, :, None])
                      | ((kj[:, None, :] == ckc[:, :, None])
                         & (cj[:, None, :] < ccc[:, :, None]))).astype(
                             jnp.int32)
                rank = rank + jnp.sum(gt, axis=2)
            outpos = jax.lax.broadcasted_iota(jnp.int32, (1, cap), 1)
            vals_acc = jnp.zeros((row_tile, cap), jnp.float32)
            idxf_acc = jnp.zeros((row_tile, cap), jnp.float32)
            for jc in range(cap // rk_chunk):
                rj = rank[:, jc * rk_chunk:(jc + 1) * rk_chunk]
                vjf = cv[:, jc * rk_chunk:(jc + 1) * rk_chunk].astype(
                    jnp.float32)
                kj = ck[:, jc * rk_chunk:(jc + 1) * rk_chunk]
                cjf = cc[:, jc * rk_chunk:(jc + 1) * rk_chunk].astype(
                    jnp.float32)
                m = (rj[:, :, None] == outpos[:, None, :]).astype(
                    jnp.float32) * vjf[:, :, None]
                vals_acc = vals_acc + jnp.sum(
                    m * _key_inv(kj)[:, :, None], axis=1)
                idxf_acc = idxf_acc + jnp.sum(m * cjf[:, :, None], axis=1)
            vals_ref[...] = vals_acc
            idx_ref[...] = idxf_acc.astype(jnp.int32)

    @jax.jit
    def run(X, all_rep, n_recos):
        X_rep = jnp.take(all_rep, X, axis=0)
        X_agg = jnp.mean(X_rep, axis=1)
        item_rep = all_rep[1:-1]
        item_pad = jnp.pad(item_rep, ((0, padc - vocab), (0, 0)))
        grid = (b // row_tile, nt)
        vals, idx = pl.pallas_call(
            body,
            grid=grid,
            in_specs=[
                pl.BlockSpec((row_tile, l), lambda i, j: (i, 0)),
                pl.BlockSpec((row_tile, d), lambda i, j: (i, 0)),
                pl.BlockSpec((col_tile, d), lambda i, j: (j, 0)),
            ],
            out_specs=[
                pl.BlockSpec((row_tile, cap), lambda i, j: (i, 0)),
                pl.BlockSpec((row_tile, cap), lambda i, j: (i, 0)),
            ],
            out_shape=[
                jax.ShapeDtypeStruct((b, cap), jnp.float32),
                jax.ShapeDtypeStruct((b, cap), jnp.int32),
            ],
            scratch_shapes=[
                pltpu.VMEM((row_tile, padc), jnp.int32),
                pltpu.VMEM((row_tile, padc), jnp.int32),
                pltpu.VMEM((row_tile, nsl), jnp.int32),
                pltpu.VMEM((row_tile, nsl), jnp.int32),
            ],
        )(X, X_agg, item_pad)
        vals = vals[:, :k] + (jnp.asarray(n_recos) - k).astype(jnp.float32)
        return vals, idx[:, :k]

    return run


_kernel_impl = None


def kernel(X, all_rep, n_recos):
    global _kernel_impl
    if _kernel_impl is None:
        _kernel_impl = _make_kernel(
            vocab=100000, d=64, b=1024, l=50, k=500,
            row_tile=16, col_tile=1024, cap=512)
    return _kernel_impl(X, all_rep, n_recos)


# speedup vs baseline: 5.9699x; 2.8841x over previous
import functools

import jax
import jax.numpy as jnp
from jax.experimental import pallas as pl
from jax.experimental.pallas import tpu as pltpu

MININT = -(2**31)


def _key_fwd(s):
    b = jax.lax.bitcast_convert_type(s, jnp.int32)
    return jnp.where(b >= 0, b, jnp.bitwise_not(b) ^ MININT)


def _key_inv(k):
    b = jnp.where(k >= 0, k, jnp.bitwise_not(k ^ MININT))
    return jax.lax.bitcast_convert_type(b, jnp.float32)


def _make_kernel(vocab, d, b, l, k, row_tile, col_tile, cap, n_bs_tiles):
    padc = ((vocab + col_tile - 1) // col_tile) * col_tile
    nt = padc // col_tile
    chunk = 128
    nch = padc // chunk
    slots = 16
    cpt = (col_tile // chunk) * slots
    nsl = nch * slots
    sl_bits = max(1, (nsl - 1).bit_length())
    rk_chunk = 64
    assert padc % n_bs_tiles == 0
    bs_tile = padc // n_bs_tiles

    def body(x_ref, xagg_ref, item_ref, vals_ref, idx_ref,
             ks_ref, ks2_ref, co2_ref):
        j = pl.program_id(1)
        scores = jax.lax.dot_general(
            xagg_ref[...], item_ref[...], (((1,), (1,)), ((), ())),
            preferred_element_type=jnp.float32,
        )
        col = j * col_tile + jax.lax.broadcasted_iota(jnp.int32, (1, col_tile), 1)
        xids = x_ref[...] - 1
        for ll in range(l):
            scores = jnp.where(col == xids[:, ll][:, None], -1.0e3, scores)
        scores = jnp.where(col >= vocab, -jnp.inf, scores)
        ks_ref[:, pl.ds(j * col_tile, col_tile)] = _key_fwd(scores)

        @pl.when(j == nt - 1)
        def _():
            lane_t = jax.lax.broadcasted_iota(jnp.int32, (1, col_tile), 1)
            lane_c = lane_t % chunk
            kk = jnp.int32(k)

            def bit_body(bi, t):
                trial = t + (jnp.int32(1) << (31 - bi))
                cnt = jnp.zeros((row_tile, 1), jnp.int32)
                for ti in range(n_bs_tiles):
                    kst = ks_ref[:, ti * bs_tile:(ti + 1) * bs_tile]
                    cnt = cnt + jnp.sum((kst >= trial).astype(jnp.int32),
                                        axis=1, keepdims=True)
                return jnp.where(cnt >= kk, trial, t)

            t = jax.lax.fori_loop(
                0, 32, bit_body,
                jnp.full((row_tile, 1), MININT, jnp.int32))

            def comp_body(ti, _):
                ksv = ks_ref[:, pl.ds(ti * col_tile, col_tile)]
                sel = ksv >= t
                p = sel.astype(jnp.int32)
                for bbit in range(7):
                    s = 1 << bbit
                    p = p + jnp.where(lane_c >= s, jnp.roll(p, s, axis=1), 0)
                dv = jnp.where(sel, (lane_c << 8) | (lane_c - (p - 1)), -1)
                for bbit in range(7):
                    s = 1 << bbit
                    kM = jnp.roll(ksv, -s, axis=1)
                    dM = jnp.roll(dv, -s, axis=1)
                    guard = lane_c <= (chunk - 1 - s)
                    take = guard & (dM >= 0) & ((dM & s) != 0)
                    moved = (dv >= 0) & ((dv & s) != 0)
                    ksv = jnp.where(take, kM, ksv)
                    dv = jnp.where(take, dM - s,
                                   jnp.where(moved, -1, dv))
                ks3 = ksv.reshape(row_tile, col_tile // chunk, chunk)
                d3 = dv.reshape(row_tile, col_tile // chunk, chunk)
                kslot = ks3[:, :, :slots].reshape(row_tile, cpt)
                dslot = d3[:, :, :slots].reshape(row_tile, cpt)
                valid = (dslot >= 0) & ((dslot & 127) == 0)
                chk = jax.lax.broadcasted_iota(
                    jnp.int32, (1, cpt), 1) // slots
                colv = ti * col_tile + chk * chunk + (dslot >> 8)
                ks2_ref[:, pl.ds(ti * cpt, cpt)] = kslot
                co2_ref[:, pl.ds(ti * cpt, cpt)] = jnp.where(valid, colv, -1)
                return _

            jax.lax.fori_loop(0, nt, comp_body, jnp.int32(0))

            lane2 = jax.lax.broadcasted_iota(jnp.int32, (1, nsl), 1)
            v2 = (co2_ref[...] >= 0).astype(jnp.int32)
            p2 = v2
            for bbit in range(sl_bits):
                s = 1 << bbit
                p2 = p2 + jnp.where(lane2 >= s, jnp.roll(p2, s, axis=1), 0)
            dv = jnp.where(v2 != 0, lane2 - (p2 - 1), -1)
            ksv = ks2_ref[...]
            cov = co2_ref[...]
            for bbit in range(sl_bits):
                s = 1 << bbit
                kM = jnp.roll(ksv, -s, axis=1)
                cM = jnp.roll(cov, -s, axis=1)
                dM = jnp.roll(dv, -s, axis=1)
                guard = lane2 <= (nsl - 1 - s)
                take = guard & (dM >= 0) & ((dM & s) != 0)
                moved = (dv >= 0) & ((dv & s) != 0)
                ksv = jnp.where(take, kM, ksv)
                cov = jnp.where(take, cM, cov)
                dv = jnp.where(take, dM - s, jnp.where(moved, -1, dv))
            ck = ksv[:, :cap]
            cc = cov[:, :cap]
            cv = dv[:, :cap] == 0

            ckc = jnp.where(cv, ck, MININT)
            ccc = jnp.where(cv, cc, jnp.int32(2**30))
            rank = jnp.zeros((row_tile, cap), jnp.int32)
            for jc in range(cap // rk_chunk):
                kj = ckc[:, jc * rk_chunk:(jc + 1) * rk_chunk]
                cj = ccc[:, jc * rk_chunk:(jc + 1) * rk_chunk]
                gt = ((kj[:, None, :] > ckc[:, :, None])
                      | ((kj[:, None, :] == ckc[:, :, None])
                         & (cj[:, None, :] < ccc[:, :, None]))).astype(
                             jnp.int32)
                rank = rank + jnp.sum(gt, axis=2)
            outpos = jax.lax.broadcasted_iota(jnp.int32, (1, cap), 1)
            vals_acc = jnp.zeros((row_tile, cap), jnp.float32)
            idxf_acc = jnp.zeros((row_tile, cap), jnp.float32)
            for jc in range(cap // rk_chunk):
                rj = rank[:, jc * rk_chunk:(jc + 1) * rk_chunk]
                vjf = cv[:, jc * rk_chunk:(jc + 1) * rk_chunk].astype(
                    jnp.float32)
                kj = ck[:, jc * rk_chunk:(jc + 1) * rk_chunk]
                cjf = cc[:, jc * rk_chunk:(jc + 1) * rk_chunk].astype(
                    jnp.float32)
                m = (rj[:, :, None] == outpos[:, None, :]).astype(
                    jnp.float32) * vjf[:, :, None]
                vals_acc = vals_acc + jnp.sum(
                    m * _key_inv(kj)[:, :, None], axis=1)
                idxf_acc = idxf_acc + jnp.sum(m * cjf[:, :, None], axis=1)
            vals_ref[...] = vals_acc
            idx_ref[...] = idxf_acc.astype(jnp.int32)

    @jax.jit
    def run(X, all_rep, n_recos):
        X_rep = jnp.take(all_rep, X, axis=0)
        X_agg = jnp.mean(X_rep, axis=1)
        item_rep = all_rep[1:-1]
        item_pad = jnp.pad(item_rep, ((0, padc - vocab), (0, 0)))
        grid = (b // row_tile, nt)
        vals, idx = pl.pallas_call(
            body,
            grid=grid,
            in_specs=[
                pl.BlockSpec((row_tile, l), lambda i, j: (i, 0)),
                pl.BlockSpec((row_tile, d), lambda i, j: (i, 0)),
                pl.BlockSpec((col_tile, d), lambda i, j: (j, 0)),
            ],
            out_specs=[
                pl.BlockSpec((row_tile, cap), lambda i, j: (i, 0)),
                pl.BlockSpec((row_tile, cap), lambda i, j: (i, 0)),
            ],
            out_shape=[
                jax.ShapeDtypeStruct((b, cap), jnp.float32),
                jax.ShapeDtypeStruct((b, cap), jnp.int32),
            ],
            scratch_shapes=[
                pltpu.VMEM((row_tile, padc), jnp.int32),
                pltpu.VMEM((row_tile, nsl), jnp.int32),
                pltpu.VMEM((row_tile, nsl), jnp.int32),
            ],
        )(X, X_agg, item_pad)
        vals = vals[:, :k] + (jnp.asarray(n_recos) - k).astype(jnp.float32)
        return vals, idx[:, :k]

    return run


_kernel_impl = None


def kernel(X, all_rep, n_recos):
    global _kernel_impl
    if _kernel_impl is None:
        _kernel_impl = _make_kernel(
            vocab=100000, d=64, b=1024, l=50, k=500,
            row_tile=32, col_tile=2048, cap=512, n_bs_tiles=14)
    return _kernel_impl(X, all_rep, n_recos)


# speedup vs baseline: 6.1594x; 1.0317x over previous
import functools

import jax
import jax.numpy as jnp
from jax.experimental import pallas as pl
from jax.experimental.pallas import tpu as pltpu

MININT = -(2**31)


def _key_fwd(s):
    b = jax.lax.bitcast_convert_type(s, jnp.int32)
    return jnp.where(b >= 0, b, jnp.bitwise_not(b) ^ MININT)


def _key_inv(k):
    b = jnp.where(k >= 0, k, jnp.bitwise_not(k ^ MININT))
    return jax.lax.bitcast_convert_type(b, jnp.float32)


def _make_kernel(vocab, d, b, l, k, row_tile, col_tile, cap, n_bs_tiles):
    padc = ((vocab + col_tile - 1) // col_tile) * col_tile
    nt = padc // col_tile
    chunk = 128
    nch = padc // chunk
    slots = 16
    cpt = (col_tile // chunk) * slots
    nsl = nch * slots
    sl_bits = max(1, (nsl - 1).bit_length())
    rk_chunk = 64
    assert padc % n_bs_tiles == 0
    bs_tile = padc // n_bs_tiles

    def body(x_ref, xagg_ref, item_ref, vals_ref, idx_ref,
             ks_ref, ks2_ref, co2_ref):
        j = pl.program_id(1)
        scores = jax.lax.dot_general(
            xagg_ref[...], item_ref[...], (((1,), (1,)), ((), ())),
            preferred_element_type=jnp.float32,
        )
        col = j * col_tile + jax.lax.broadcasted_iota(jnp.int32, (1, col_tile), 1)
        xids = x_ref[...] - 1
        for ll in range(l):
            scores = jnp.where(col == xids[:, ll][:, None], -1.0e3, scores)
        scores = jnp.where(col >= vocab, -jnp.inf, scores)
        ks_ref[:, pl.ds(j * col_tile, col_tile)] = _key_fwd(scores)

        @pl.when(j == nt - 1)
        def _():
            lane_t = jax.lax.broadcasted_iota(jnp.int32, (1, col_tile), 1)
            lane_c = lane_t % chunk
            kk = jnp.int32(k)

            def bit_body(bi, t):
                trial = t + (jnp.int32(1) << (31 - bi))
                cnt = jnp.zeros((row_tile, 1), jnp.int32)
                for ti in range(n_bs_tiles):
                    kst = ks_ref[:, ti * bs_tile:(ti + 1) * bs_tile]
                    cnt = cnt + jnp.sum((kst >= trial).astype(jnp.int32),
                                        axis=1, keepdims=True)
                return jnp.where(cnt >= kk, trial, t)

            t = jax.lax.fori_loop(
                0, 24, bit_body,
                jnp.full((row_tile, 1), MININT, jnp.int32))

            def comp_body(ti, _):
                ksv = ks_ref[:, pl.ds(ti * col_tile, col_tile)]
                sel = ksv >= t
                p = sel.astype(jnp.int32)
                for bbit in range(7):
                    s = 1 << bbit
                    p = p + jnp.where(lane_c >= s, jnp.roll(p, s, axis=1), 0)
                dv = jnp.where(sel, (lane_c << 8) | (lane_c - (p - 1)), -1)
                for bbit in range(7):
                    s = 1 << bbit
                    kM = jnp.roll(ksv, -s, axis=1)
                    dM = jnp.roll(dv, -s, axis=1)
                    guard = lane_c <= (chunk - 1 - s)
                    take = guard & (dM >= 0) & ((dM & s) != 0)
                    moved = (dv >= 0) & ((dv & s) != 0)
                    ksv = jnp.where(take, kM, ksv)
                    dv = jnp.where(take, dM - s,
                                   jnp.where(moved, -1, dv))
                ks3 = ksv.reshape(row_tile, col_tile // chunk, chunk)
                d3 = dv.reshape(row_tile, col_tile // chunk, chunk)
                kslot = ks3[:, :, :slots].reshape(row_tile, cpt)
                dslot = d3[:, :, :slots].reshape(row_tile, cpt)
                valid = (dslot >= 0) & ((dslot & 127) == 0)
                chk = jax.lax.broadcasted_iota(
                    jnp.int32, (1, cpt), 1) // slots
                colv = ti * col_tile + chk * chunk + (dslot >> 8)
                ks2_ref[:, pl.ds(ti * cpt, cpt)] = kslot
                co2_ref[:, pl.ds(ti * cpt, cpt)] = jnp.where(valid, colv, -1)
                return _

            jax.lax.fori_loop(0, nt, comp_body, jnp.int32(0))

            lane2 = jax.lax.broadcasted_iota(jnp.int32, (1, nsl), 1)
            v2 = (co2_ref[...] >= 0).astype(jnp.int32)
            p2 = v2
            for bbit in range(sl_bits):
                s = 1 << bbit
                p2 = p2 + jnp.where(lane2 >= s, jnp.roll(p2, s, axis=1), 0)
            dv = jnp.where(v2 != 0, lane2 - (p2 - 1), -1)
            ksv = ks2_ref[...]
            cov = co2_ref[...]
            for bbit in range(sl_bits):
                s = 1 << bbit
                kM = jnp.roll(ksv, -s, axis=1)
                cM = jnp.roll(cov, -s, axis=1)
                dM = jnp.roll(dv, -s, axis=1)
                guard = lane2 <= (nsl - 1 - s)
                take = guard & (dM >= 0) & ((dM & s) != 0)
                moved = (dv >= 0) & ((dv & s) != 0)
                ksv = jnp.where(take, kM, ksv)
                cov = jnp.where(take, cM, cov)
                dv = jnp.where(take, dM - s, jnp.where(moved, -1, dv))
            ck = ksv[:, :cap]
            cc = cov[:, :cap]
            cv = dv[:, :cap] == 0

            ckc = jnp.where(cv, ck, MININT)
            ccc = jnp.where(cv, cc, jnp.int32(2**30))
            rank = jnp.zeros((row_tile, cap), jnp.int32)
            for jc in range(cap // rk_chunk):
                kj = ckc[:, jc * rk_chunk:(jc + 1) * rk_chunk]
                cj = ccc[:, jc * rk_chunk:(jc + 1) * rk_chunk]
                gt = ((kj[:, None, :] > ckc[:, :, None])
                      | ((kj[:, None, :] == ckc[:, :, None])
                         & (cj[:, None, :] < ccc[:, :, None]))).astype(
                             jnp.int32)
                rank = rank + jnp.sum(gt, axis=2)
            outpos = jax.lax.broadcasted_iota(jnp.int32, (1, cap), 1)
            vals_acc = jnp.zeros((row_tile, cap), jnp.float32)
            idxf_acc = jnp.zeros((row_tile, cap), jnp.float32)
            for jc in range(cap // rk_chunk):
                rj = rank[:, jc * rk_chunk:(jc + 1) * rk_chunk]
                vjf = cv[:, jc * rk_chunk:(jc + 1) * rk_chunk].astype(
                    jnp.float32)
                kj = ck[:, jc * rk_chunk:(jc + 1) * rk_chunk]
                cjf = cc[:, jc * rk_chunk:(jc + 1) * rk_chunk].astype(
                    jnp.float32)
                m = (rj[:, :, None] == outpos[:, None, :]).astype(
                    jnp.float32) * vjf[:, :, None]
                vals_acc = vals_acc + jnp.sum(
                    m * _key_inv(kj)[:, :, None], axis=1)
                idxf_acc = idxf_acc + jnp.sum(m * cjf[:, :, None], axis=1)
            vals_ref[...] = vals_acc
            idx_ref[...] = idxf_acc.astype(jnp.int32)

    @jax.jit
    def run(X, all_rep, n_recos):
        X_rep = jnp.take(all_rep, X, axis=0)
        X_agg = jnp.mean(X_rep, axis=1)
        item_rep = all_rep[1:-1]
        item_pad = jnp.pad(item_rep, ((0, padc - vocab), (0, 0)))
        grid = (b // row_tile, nt)
        vals, idx = pl.pallas_call(
            body,
            grid=grid,
            in_specs=[
                pl.BlockSpec((row_tile, l), lambda i, j: (i, 0)),
                pl.BlockSpec((row_tile, d), lambda i, j: (i, 0)),
                pl.BlockSpec((col_tile, d), lambda i, j: (j, 0)),
            ],
            out_specs=[
                pl.BlockSpec((row_tile, cap), lambda i, j: (i, 0)),
                pl.BlockSpec((row_tile, cap), lambda i, j: (i, 0)),
            ],
            out_shape=[
                jax.ShapeDtypeStruct((b, cap), jnp.float32),
                jax.ShapeDtypeStruct((b, cap), jnp.int32),
            ],
            scratch_shapes=[
                pltpu.VMEM((row_tile, padc), jnp.int32),
                pltpu.VMEM((row_tile, nsl), jnp.int32),
                pltpu.VMEM((row_tile, nsl), jnp.int32),
            ],
        )(X, X_agg, item_pad)
        vals = vals[:, :k] + (jnp.asarray(n_recos) - k).astype(jnp.float32)
        return vals, idx[:, :k]

    return run


_kernel_impl = None


def kernel(X, all_rep, n_recos):
    global _kernel_impl
    if _kernel_impl is None:
        _kernel_impl = _make_kernel(
            vocab=100000, d=64, b=1024, l=50, k=500,
            row_tile=32, col_tile=2048, cap=512, n_bs_tiles=14)
    return _kernel_impl(X, all_rep, n_recos)


# speedup vs baseline: 6.1607x; 1.0002x over previous
import jax
import jax.numpy as jnp
from jax.experimental import pallas as pl
from jax.experimental.pallas import tpu as pltpu

MININT = -(2**31)


def _key_fwd(s):
    b = jax.lax.bitcast_convert_type(s, jnp.int32)
    return jnp.where(b >= 0, b, jnp.bitwise_not(b) ^ MININT)


def _key_inv(k):
    b = jnp.where(k >= 0, k, jnp.bitwise_not(k ^ MININT))
    return jax.lax.bitcast_convert_type(b, jnp.float32)


def _make_kernel(vocab, d, b, l, k, row_tile, col_tile, cap, n_bs_tiles):
    padc = ((vocab + col_tile - 1) // col_tile) * col_tile
    nt = padc // col_tile
    chunk = 128
    nch = padc // chunk
    slots = 16
    cpt = (col_tile // chunk) * slots
    nsl = nch * slots
    sl_bits = max(1, (nsl - 1).bit_length())
    rk_chunk = 64
    assert padc % n_bs_tiles == 0
    bs_tile = padc // n_bs_tiles

    def body(x_ref, xagg_ref, item_ref, vals_ref, idx_ref,
             ks_ref, ks2_ref, co2_ref):
        j = pl.program_id(1)
        scores = jax.lax.dot_general(
            xagg_ref[...], item_ref[...], (((1,), (1,)), ((), ())),
            preferred_element_type=jnp.float32,
        )
        col = j * col_tile + jax.lax.broadcasted_iota(jnp.int32, (1, col_tile), 1)
        xids = x_ref[...] - 1
        for ll in range(l):
            scores = jnp.where(col == xids[:, ll][:, None], -1.0e3, scores)
        scores = jnp.where(col >= vocab, -jnp.inf, scores)
        ks_ref[:, pl.ds(j * col_tile, col_tile)] = _key_fwd(scores)

        @pl.when(j == nt - 1)
        def _():
            lane_t = jax.lax.broadcasted_iota(jnp.int32, (1, col_tile), 1)
            lane_c = lane_t % chunk
            kk = jnp.int32(k)

            def bit_body(bi, t):
                trial = t + (jnp.int32(1) << (31 - bi))
                cnt = jnp.zeros((row_tile, 1), jnp.int32)
                for ti in range(n_bs_tiles):
                    kst = ks_ref[:, ti * bs_tile:(ti + 1) * bs_tile]
                    cnt = cnt + jnp.sum((kst >= trial).astype(jnp.int32),
                                        axis=1, keepdims=True)
                return jnp.where(cnt >= kk, trial, t)

            t = jax.lax.fori_loop(
                0, 24, bit_body,
                jnp.full((row_tile, 1), MININT, jnp.int32))

            def comp_body(ti, _):
                ksv = ks_ref[:, pl.ds(ti * col_tile, col_tile)]
                sel = ksv >= t
                p = sel.astype(jnp.int32)
                for bbit in range(7):
                    s = 1 << bbit
                    p = p + jnp.where(lane_c >= s, jnp.roll(p, s, axis=1), 0)
                dv = jnp.where(sel, (lane_c << 8) | (lane_c - (p - 1)), -1)
                for bbit in range(7):
                    s = 1 << bbit
                    kM = jnp.roll(ksv, -s, axis=1)
                    dM = jnp.roll(dv, -s, axis=1)
                    guard = lane_c <= (chunk - 1 - s)
                    take = guard & (dM >= 0) & ((dM & s) != 0)
                    moved = (dv >= 0) & ((dv & s) != 0)
                    ksv = jnp.where(take, kM, ksv)
                    dv = jnp.where(take, dM - s,
                                   jnp.where(moved, -1, dv))
                ks3 = ksv.reshape(row_tile, col_tile // chunk, chunk)
                d3 = dv.reshape(row_tile, col_tile // chunk, chunk)
                kslot = ks3[:, :, :slots].reshape(row_tile, cpt)
                dslot = d3[:, :, :slots].reshape(row_tile, cpt)
                valid = (dslot >= 0) & ((dslot & 127) == 0)
                chk = jax.lax.broadcasted_iota(
                    jnp.int32, (1, cpt), 1) // slots
                colv = ti * col_tile + chk * chunk + (dslot >> 8)
                ks2_ref[:, pl.ds(ti * cpt, cpt)] = kslot
                co2_ref[:, pl.ds(ti * cpt, cpt)] = jnp.where(valid, colv, -1)
                return _

            jax.lax.fori_loop(0, nt, comp_body, jnp.int32(0))

            lane2 = jax.lax.broadcasted_iota(jnp.int32, (1, nsl), 1)
            v2 = (co2_ref[...] >= 0).astype(jnp.int32)
            p2 = v2
            for bbit in range(sl_bits):
                s = 1 << bbit
                p2 = p2 + jnp.where(lane2 >= s, jnp.roll(p2, s, axis=1), 0)
            dv = jnp.where(v2 != 0, lane2 - (p2 - 1), -1)
            ksv = ks2_ref[...]
            cov = co2_ref[...]
            for bbit in range(sl_bits):
                s = 1 << bbit
                kM = jnp.roll(ksv, -s, axis=1)
                cM = jnp.roll(cov, -s, axis=1)
                dM = jnp.roll(dv, -s, axis=1)
                guard = lane2 <= (nsl - 1 - s)
                take = guard & (dM >= 0) & ((dM & s) != 0)
                moved = (dv >= 0) & ((dv & s) != 0)
                ksv = jnp.where(take, kM, ksv)
                cov = jnp.where(take, cM, cov)
                dv = jnp.where(take, dM - s, jnp.where(moved, -1, dv))
            ck = ksv[:, :cap]
            cc = cov[:, :cap]
            cv = dv[:, :cap] == 0

            ckc = jnp.where(cv, ck, MININT)
            ccc = jnp.where(cv, cc, jnp.int32(2**30))
            rank = jnp.zeros((row_tile, cap), jnp.int32)
            for jc in range(cap // rk_chunk):
                kj = ckc[:, jc * rk_chunk:(jc + 1) * rk_chunk]
                cj = ccc[:, jc * rk_chunk:(jc + 1) * rk_chunk]
                gt = ((kj[:, None, :] > ckc[:, :, None])
                      | ((kj[:, None, :] == ckc[:, :, None])
                         & (cj[:, None, :] < ccc[:, :, None]))).astype(
                             jnp.int32)
                rank = rank + jnp.sum(gt, axis=2)
            outpos = jax.lax.broadcasted_iota(jnp.int32, (1, cap), 1)
            vals_acc = jnp.zeros((row_tile, cap), jnp.float32)
            idxf_acc = jnp.zeros((row_tile, cap), jnp.float32)
            for jc in range(cap // rk_chunk):
                rj = rank[:, jc * rk_chunk:(jc + 1) * rk_chunk]
                vjf = cv[:, jc * rk_chunk:(jc + 1) * rk_chunk].astype(
                    jnp.float32)
                kj = ck[:, jc * rk_chunk:(jc + 1) * rk_chunk]
                cjf = cc[:, jc * rk_chunk:(jc + 1) * rk_chunk].astype(
                    jnp.float32)
                m = (rj[:, :, None] == outpos[:, None, :]).astype(
                    jnp.float32) * vjf[:, :, None]
                vals_acc = vals_acc + jnp.sum(
                    m * _key_inv(kj)[:, :, None], axis=1)
                idxf_acc = idxf_acc + jnp.sum(m * cjf[:, :, None], axis=1)
            vals_ref[...] = vals_acc
            idx_ref[...] = idxf_acc.astype(jnp.int32)

    @jax.jit
    def run(X, all_rep, n_recos):
        X_rep = jnp.take(all_rep, X, axis=0)
        X_agg = jnp.mean(X_rep, axis=1)
        item_rep = all_rep[1:-1]
        item_pad = jnp.pad(item_rep, ((0, padc - vocab), (0, 0)))
        grid = (b // row_tile, nt)
        vals, idx = pl.pallas_call(
            body,
            grid=grid,
            in_specs=[
                pl.BlockSpec((row_tile, l), lambda i, j: (i, 0)),
                pl.BlockSpec((row_tile, d), lambda i, j: (i, 0)),
                pl.BlockSpec((col_tile, d), lambda i, j: (j, 0)),
            ],
            out_specs=[
                pl.BlockSpec((row_tile, cap), lambda i, j: (i, 0)),
                pl.BlockSpec((row_tile, cap), lambda i, j: (i, 0)),
            ],
            out_shape=[
                jax.ShapeDtypeStruct((b, cap), jnp.float32),
                jax.ShapeDtypeStruct((b, cap), jnp.int32),
            ],
            scratch_shapes=[
                pltpu.VMEM((row_tile, padc), jnp.int32),
                pltpu.VMEM((row_tile, nsl), jnp.int32),
                pltpu.VMEM((row_tile, nsl), jnp.int32),
            ],
        )(X, X_agg, item_pad)
        vals = vals[:, :k] + (jnp.asarray(n_recos) - k).astype(jnp.float32)
        return vals, idx[:, :k]

    return run


_kernel_impl = None


def kernel(X, all_rep, n_recos):
    global _kernel_impl
    if _kernel_impl is None:
        _kernel_impl = _make_kernel(
            vocab=100000, d=64, b=1024, l=50, k=500,
            row_tile=32, col_tile=2048, cap=512, n_bs_tiles=14)
    return _kernel_impl(X, all_rep, n_recos)
